# baseline (speedup 1.0000x reference)
import functools
import math

import jax
import jax.numpy as jnp
from jax import lax
from jax.experimental import pallas as pl
from jax.experimental.pallas import tpu as pltpu
from jax.experimental.pallas import tpu_sc as plsc

_NBUF = 8
_NFLY = 4


def _emb_lookup(idx, table, n_per, chunk, scale):
    N, = idx.shape
    V, D = table.shape
    n_chunks = n_per // chunk
    n_pairs = _NBUF // 2
    assert n_chunks % _NBUF == 0 and n_chunks >= 2 * _NBUF
    nc = plsc.get_sparse_core_info().num_cores
    mesh = plsc.VectorSubcoreMesh(core_axis_name="c", subcore_axis_name="s")

    @functools.partial(
        pl.kernel,
        mesh=mesh,
        out_type=jax.ShapeDtypeStruct((N, D), jnp.float32),
        scratch_types=[
            pltpu.VMEM((n_per,), jnp.int32),
            pltpu.VMEM((_NBUF * chunk, D), jnp.float32),
            pltpu.SemaphoreType.DMA((_NBUF,)),
            pltpu.SemaphoreType.DMA((n_pairs,)),
        ],
        compiler_params=pltpu.CompilerParams(use_tc_tiling_on_sc=False),
    )
    def body(idx_hbm, table_hbm, out_hbm, idx_v, rows_v, gsem, ssem):
        wid = lax.axis_index("s") * nc + lax.axis_index("c")
        base = wid * n_per

        def start_gather(g, b):
            pltpu.async_copy(table_hbm.at[idx_v.at[pl.ds(g * chunk, chunk)]],
                             rows_v.at[pl.ds(b * chunk, chunk)], gsem.at[b])

        def wait_gather(g, b):
            pltpu.make_async_copy(table_hbm.at[idx_v.at[pl.ds(g * chunk, chunk)]],
                                  rows_v.at[pl.ds(b * chunk, chunk)],
                                  gsem.at[b]).wait()

        def scale_rows(b):
            def grp_body(r8, carry):
                for r in range(8):
                    for j in range(D // 16):
                        sl = pl.ds(j * 16, 16)
                        rows_v[b * chunk + r8 * 8 + r, sl] = (
                            rows_v[b * chunk + r8 * 8 + r, sl] * scale)
                return carry

            lax.fori_loop(0, chunk // 8, grp_body, 0)

        def start_scatter(g, b):
            start = base + (g - 1) * chunk
            pltpu.async_copy(rows_v.at[pl.ds((b - 1) * chunk, 2 * chunk)],
                             out_hbm.at[pl.ds(start, 2 * chunk)],
                             ssem.at[b // 2])

        def wait_scatter(g, b):
            start = base + (g - 1) * chunk
            pltpu.make_async_copy(rows_v.at[pl.ds((b - 1) * chunk, 2 * chunk)],
                                  out_hbm.at[pl.ds(start, 2 * chunk)],
                                  ssem.at[b // 2]).wait()

        pltpu.sync_copy(idx_hbm.at[pl.ds(base, n_per)], idx_v)
        for b in range(_NFLY):
            start_gather(b, b)

        def grp_body(p, carry):
            for b in range(_NBUF):
                g = p * _NBUF + b
                bn = (b + _NFLY) % _NBUF

                @pl.when(g + _NFLY < n_chunks)
                def _():
                    if bn % 2 == 0:
                        @pl.when(g >= _NFLY)
                        def _():
                            wait_scatter(g - _NFLY + 1, bn + 1)

                    start_gather(g + _NFLY, bn)

                wait_gather(g, b)
                scale_rows(b)
                if b % 2 == 1:
                    start_scatter(g, b)

            return carry

        lax.fori_loop(0, n_chunks // _NBUF, grp_body, 0)
        for k in range(n_pairs):
            b = 2 * k + 1
            g = n_chunks - _NBUF + b
            wait_scatter(g, b)

    return body(idx, table)


def kernel(tokens, embedding):
    B, S = tokens.shape
    V, D = embedding.shape
    N = B * S
    idx = tokens.reshape(N).astype(jnp.int32)
    info = plsc.get_sparse_core_info()
    n_workers = info.num_cores * info.num_subcores
    n_per = N // n_workers
    out = _emb_lookup(idx, embedding, n_per, 128, math.sqrt(D))
    return out.reshape(B, S, D)
